# Initial kernel scaffold; baseline (speedup 1.0000x reference)
import jax
import jax.numpy as jnp
from jax.experimental import pallas as pl


def kernel(scores):
    raise NotImplementedError("write your pallas kernel here")


# speedup vs baseline: 7.6424x; 7.6424x over previous
import functools

import jax
import jax.numpy as jnp
from jax.experimental import pallas as pl
from jax.experimental.pallas import tpu as pltpu

_ROWS = 256


def _adj_kernel(s_ref, o_ref, *, n, other_k):
    rb = pl.program_id(1)
    r = s_ref.shape[1]
    s = s_ref[0]

    col = jax.lax.broadcasted_iota(jnp.int32, (r, n), 1)
    row = jax.lax.broadcasted_iota(jnp.int32, (r, n), 0) + rb * r
    diag = col == row

    m = jnp.max(s, axis=-1, keepdims=True)
    e = jnp.exp(s - m)
    z = jnp.sum(e, axis=-1, keepdims=True)
    p = e / z
    p_others = jnp.where(diag, 0.0, p)

    work = p_others
    mask = jnp.zeros((r, n), dtype=jnp.bool_)
    for _ in range(other_k):
        mx = jnp.max(work, axis=-1, keepdims=True)
        cand = jnp.where(work == mx, col, n)
        amin = jnp.min(cand, axis=-1, keepdims=True)
        sel = col == amin
        mask = jnp.logical_or(mask, sel)
        work = jnp.where(sel, -1.0, work)

    sp = jnp.where(mask, p_others, 0.0)
    sp = sp / (jnp.sum(sp, axis=-1, keepdims=True) + 1e-8)

    o_ref[0, :, 0, :] = jnp.where(diag, 1.0, 0.0)
    o_ref[0, :, 1:, :] = jnp.broadcast_to(sp[:, None, :], (r, other_k, n))


def kernel(scores):
    b, n, _ = scores.shape
    total_k = 8
    other_k = total_k - 1
    rows = _ROWS
    grid = (b, n // rows)
    out = pl.pallas_call(
        functools.partial(_adj_kernel, n=n, other_k=other_k),
        grid=grid,
        in_specs=[
            pl.BlockSpec((1, rows, n), lambda bi, ri: (bi, ri, 0)),
        ],
        out_specs=pl.BlockSpec(
            (1, rows, total_k, n), lambda bi, ri: (bi, ri, 0, 0)
        ),
        out_shape=jax.ShapeDtypeStruct((b, n, total_k, n), scores.dtype),
    )(scores)
    return out


# speedup vs baseline: 11.4294x; 1.4955x over previous
import functools

import jax
import jax.numpy as jnp
from jax.experimental import pallas as pl
from jax.experimental.pallas import tpu as pltpu

_ROWS = 256


def _adj_kernel(s_ref, o_ref, *, n, other_k):
    rb = pl.program_id(1)
    r = s_ref.shape[1]
    s = s_ref[0]

    col = jax.lax.broadcasted_iota(jnp.int32, (r, n), 1)
    row = jax.lax.broadcasted_iota(jnp.int32, (r, n), 0) + rb * r
    diag = col == row

    neg = jnp.float32(-3.0e38)
    work = jnp.where(diag, neg, s)
    m1 = jnp.max(work, axis=-1, keepdims=True)
    mask = jnp.zeros((r, n), dtype=jnp.bool_)
    mx = m1
    for t in range(other_k):
        sel = work == mx
        mask = jnp.logical_or(mask, sel)
        if t < other_k - 1:
            work = jnp.where(sel, neg, work)
            mx = jnp.max(work, axis=-1, keepdims=True)

    e_all = jnp.exp(jnp.minimum(s - m1, 60.0))
    z = jnp.sum(e_all, axis=-1, keepdims=True)
    e_sel = jnp.where(mask, e_all, 0.0)
    s7 = jnp.sum(e_sel, axis=-1, keepdims=True)
    sp = e_sel / (s7 + 1e-8 * z)

    o_ref[0, :, 0, :] = jnp.where(diag, 1.0, 0.0)
    o_ref[0, :, 1:, :] = jnp.broadcast_to(sp[:, None, :], (r, other_k, n))


def kernel(scores):
    b, n, _ = scores.shape
    total_k = 8
    other_k = total_k - 1
    rows = _ROWS
    grid = (b, n // rows)
    out = pl.pallas_call(
        functools.partial(_adj_kernel, n=n, other_k=other_k),
        grid=grid,
        in_specs=[
            pl.BlockSpec((1, rows, n), lambda bi, ri: (bi, ri, 0)),
        ],
        out_specs=pl.BlockSpec(
            (1, rows, total_k, n), lambda bi, ri: (bi, ri, 0, 0)
        ),
        out_shape=jax.ShapeDtypeStruct((b, n, total_k, n), scores.dtype),
    )(scores)
    return out


# speedup vs baseline: 12.9658x; 1.1344x over previous
import functools

import jax
import jax.numpy as jnp
from jax.experimental import pallas as pl
from jax.experimental.pallas import tpu as pltpu

_ROWS = 512


def _adj_kernel(s_ref, o_ref, *, n, other_k):
    rb = pl.program_id(1)
    r = s_ref.shape[1]
    s = s_ref[0]

    col = jax.lax.broadcasted_iota(jnp.int32, (r, n), 1)
    row = jax.lax.broadcasted_iota(jnp.int32, (r, n), 0) + rb * r
    diag = col == row

    neg = jnp.float32(-3.0e38)
    work = jnp.where(diag, neg, s)
    m1 = jnp.max(work, axis=-1, keepdims=True)
    mask = jnp.zeros((r, n), dtype=jnp.bool_)
    mx = m1
    for t in range(other_k):
        sel = work == mx
        mask = jnp.logical_or(mask, sel)
        if t < other_k - 1:
            work = jnp.where(sel, neg, work)
            mx = jnp.max(work, axis=-1, keepdims=True)

    e_all = jnp.exp(jnp.minimum(s - m1, 60.0))
    z = jnp.sum(e_all, axis=-1, keepdims=True)
    e_sel = jnp.where(mask, e_all, 0.0)
    s7 = jnp.sum(e_sel, axis=-1, keepdims=True)
    sp = e_sel / (s7 + 1e-8 * z)

    o_ref[0, :, 0, :] = jnp.where(diag, 1.0, 0.0)
    o_ref[0, :, 1:, :] = jnp.broadcast_to(sp[:, None, :], (r, other_k, n))


def kernel(scores):
    b, n, _ = scores.shape
    total_k = 8
    other_k = total_k - 1
    rows = _ROWS
    grid = (b, n // rows)
    out = pl.pallas_call(
        functools.partial(_adj_kernel, n=n, other_k=other_k),
        grid=grid,
        in_specs=[
            pl.BlockSpec((1, rows, n), lambda bi, ri: (bi, ri, 0)),
        ],
        out_specs=pl.BlockSpec(
            (1, rows, total_k, n), lambda bi, ri: (bi, ri, 0, 0)
        ),
        out_shape=jax.ShapeDtypeStruct((b, n, total_k, n), scores.dtype),
    )(scores)
    return out
